# Initial kernel scaffold; baseline (speedup 1.0000x reference)
import jax
import jax.numpy as jnp
from jax.experimental import pallas as pl


def kernel(inputs, codebook):
    raise NotImplementedError("write your pallas kernel here")


# speedup vs baseline: 3.0022x; 3.0022x over previous
import jax
import jax.numpy as jnp
from jax.experimental import pallas as pl
from jax.experimental.pallas import tpu as pltpu

N_EMB = 8192
DIM = 32
N_TOK = 8192
TILE = 256
GRID = N_TOK // TILE


def _vq_body(x_ref, cb_ref, loss_ref, quant_ref, soft_ref, perp_ref, idx_ref,
             cbn_ref, counts_ref, lsum_ref):
    i = pl.program_id(0)

    @pl.when(i == 0)
    def _init():
        cb = cb_ref[...]
        n = jnp.sqrt(jnp.sum(cb * cb, axis=1, keepdims=True))
        cbn_ref[...] = cb / jnp.maximum(n, 1e-12)
        counts_ref[...] = jnp.zeros_like(counts_ref)
        lsum_ref[0, 0] = 0.0

    x = x_ref[...]
    xn = x / jnp.maximum(jnp.sqrt(jnp.sum(x * x, axis=1, keepdims=True)), 1e-12)
    cbn = cbn_ref[...]
    logits = jax.lax.dot_general(xn, cbn, (((1,), (1,)), ((), ())),
                                 preferred_element_type=jnp.float32)
    d = 2.0 - 2.0 * logits
    t = -d / 0.1
    tmax = jnp.max(t, axis=1, keepdims=True)
    e = jnp.exp(t - tmax)
    soft_ref[...] = e / jnp.sum(e, axis=1, keepdims=True)

    idx = jnp.argmin(d, axis=1).astype(jnp.int32)
    idx_ref[0, 0, :] = idx

    col = jax.lax.broadcasted_iota(jnp.int32, (TILE, N_EMB), 1)
    onehot = (col == idx[:, None]).astype(jnp.float32)
    q = jax.lax.dot_general(onehot, cbn, (((1,), (0,)), ((), ())),
                            preferred_element_type=jnp.float32)
    quant_ref[...] = q
    diff = q - xn
    lsum_ref[0, 0] += jnp.sum(diff * diff)
    counts_ref[...] += jnp.sum(onehot, axis=0, keepdims=True)

    @pl.when(i == GRID - 1)
    def _fin():
        loss_ref[...] = jnp.reshape(0.25 * lsum_ref[0, 0] / (N_TOK * DIM), (1, 1))
        avg = counts_ref[...] / N_TOK
        perp_ref[...] = jnp.reshape(jnp.exp(-jnp.sum(avg * jnp.log(avg + 1e-10))), (1, 1))


def kernel(inputs, codebook):
    flat = inputs.reshape(-1, DIM)
    loss, quant, soft, perp, idx = pl.pallas_call(
        _vq_body,
        grid=(GRID,),
        in_specs=[
            pl.BlockSpec((TILE, DIM), lambda i: (i, 0)),
            pl.BlockSpec((N_EMB, DIM), lambda i: (0, 0)),
        ],
        out_specs=[
            pl.BlockSpec((1, 1), lambda i: (0, 0)),
            pl.BlockSpec((TILE, DIM), lambda i: (i, 0)),
            pl.BlockSpec((TILE, N_EMB), lambda i: (i, 0)),
            pl.BlockSpec((1, 1), lambda i: (0, 0)),
            pl.BlockSpec((1, 1, TILE), lambda i: (i, 0, 0)),
        ],
        out_shape=[
            jax.ShapeDtypeStruct((1, 1), jnp.float32),
            jax.ShapeDtypeStruct((N_TOK, DIM), jnp.float32),
            jax.ShapeDtypeStruct((N_TOK, N_EMB), jnp.float32),
            jax.ShapeDtypeStruct((1, 1), jnp.float32),
            jax.ShapeDtypeStruct((GRID, 1, TILE), jnp.int32),
        ],
        scratch_shapes=[
            pltpu.VMEM((N_EMB, DIM), jnp.float32),
            pltpu.VMEM((1, N_EMB), jnp.float32),
            pltpu.SMEM((1, 1), jnp.float32),
        ],
    )(flat, codebook)
    return (loss[0, 0], quant.reshape(inputs.shape), soft, perp[0, 0],
            idx.reshape(-1, 1))
